# baseline (speedup 1.0000x reference)
import functools

import jax
import jax.numpy as jnp
from jax import lax
from jax.experimental import pallas as pl
from jax.experimental.pallas import tpu as pltpu
from jax.experimental.pallas import tpu_sc as plsc

N_NODES = 100000
N_EDGES = 3200000
EMBED = 16

NC = 2
NS = 16
NW = NC * NS
LANES = 16

NP = 102400
NPT = NP // NS
EPW = N_EDGES // NW
W = 4000
WINDOWS = EPW // W
WA = 400
WINDOWS_A = EPW // WA
ZROWS = 400

_mesh = plsc.VectorSubcoreMesh(
    core_axis_name="c", subcore_axis_name="s", num_cores=NC, num_subcores=NS)
_sc_params = pltpu.CompilerParams(needs_layout_passes=False,
                                  use_tc_tiling_on_sc=False)


def _worker(c, s):
    return s * NC + c


@functools.partial(
    pl.kernel,
    out_type=(
        jax.ShapeDtypeStruct((N_EDGES,), jnp.float32),
        jax.ShapeDtypeStruct((NC, NP), jnp.float32),
        jax.ShapeDtypeStruct((NC, NP), jnp.float32),
    ),
    mesh=_mesh,
    scratch_types=[
        pltpu.VMEM((W,), jnp.int32),
        pltpu.VMEM((W,), jnp.int32),
        pltpu.VMEM((W,), jnp.float32),
        pltpu.VMEM((W,), jnp.float32),
        pltpu.VMEM((W,), jnp.float32),
        pltpu.VMEM((W,), jnp.float32),
        pltpu.VMEM((W,), jnp.float32),
        pltpu.VMEM((W,), jnp.float32),
        pltpu.VMEM((W,), jnp.float32),
        pltpu.VMEM((W,), jnp.float32),
        pltpu.VMEM((NPT,), jnp.float32),
        pltpu.VMEM_SHARED((NP,), jnp.float32),
        pltpu.VMEM_SHARED((NP,), jnp.float32),
        pltpu.VMEM_SHARED((NP,), jnp.float32),
        pltpu.VMEM_SHARED((NP,), jnp.float32),
        pltpu.VMEM_SHARED((NP,), jnp.float32),
        pltpu.SemaphoreType.DMA,
    ],
    compiler_params=_sc_params,
)
def _edge_weights_sc(row_hbm, col_hbm, px_hbm, py_hbm, pz_hbm,
                     ew_hbm, degp_hbm, cntp_hbm,
                     row_v, col_v, xr, yr, zr, xc, yc, zc, ew_v, one_v, zb_v,
                     spx, spy, spz, sdeg, scnt, sem):
    c = lax.axis_index("c")
    s = lax.axis_index("s")
    wid = _worker(c, s)
    nsl = pl.ds(pl.multiple_of(s * NPT, 8), NPT)

    pltpu.sync_copy(px_hbm.at[nsl], spx.at[nsl])
    pltpu.sync_copy(py_hbm.at[nsl], spy.at[nsl])
    pltpu.sync_copy(pz_hbm.at[nsl], spz.at[nsl])

    zeros16 = jnp.zeros((LANES,), jnp.float32)
    ones16 = jnp.ones((LANES,), jnp.float32)

    def zb_body(i, _):
        zb_v[pl.ds(i * LANES, LANES)] = zeros16
        return 0
    lax.fori_loop(0, NPT // LANES, zb_body, 0)

    def one_body(i, _):
        one_v[pl.ds(i * LANES, LANES)] = ones16
        return 0
    lax.fori_loop(0, W // LANES, one_body, 0)

    pltpu.sync_copy(zb_v, sdeg.at[nsl])
    pltpu.sync_copy(zb_v, scnt.at[nsl])
    plsc.subcore_barrier()

    ebase = wid * EPW

    def win(k, _):
        b = pl.multiple_of(ebase + k * W, 8)
        esl = pl.ds(b, W)
        pltpu.sync_copy(row_hbm.at[esl], row_v)
        pltpu.sync_copy(col_hbm.at[esl], col_v)
        cps = [
            pltpu.async_copy(spx.at[row_v], xr, sem),
            pltpu.async_copy(spy.at[row_v], yr, sem),
            pltpu.async_copy(spz.at[row_v], zr, sem),
            pltpu.async_copy(spx.at[col_v], xc, sem),
            pltpu.async_copy(spy.at[col_v], yc, sem),
            pltpu.async_copy(spz.at[col_v], zc, sem),
        ]
        for cp in cps:
            cp.wait()

        def chunk(j, _):
            sl = pl.ds(j * LANES, LANES)
            dx = xr[sl] - xc[sl]
            dy = yr[sl] - yc[sl]
            dz = zr[sl] - zc[sl]
            sq = dx * dx + dy * dy + dz * dz
            bits = plsc.bitcast(sq, jnp.int32)
            y = plsc.bitcast(jnp.int32(0x5F3759DF) - (bits >> 1), jnp.float32)
            y = y * (1.5 - 0.5 * sq * y * y)
            y = y * (1.5 - 0.5 * sq * y * y)
            y = y * (1.5 - 0.5 * sq * y * y)
            ew = jnp.where(sq > 0.0, sq * y, 0.0)
            ew_v[sl] = ew
            return 0
        lax.fori_loop(0, W // LANES, chunk, 0)

        pltpu.sync_copy(ew_v, ew_hbm.at[esl])
        pltpu.sync_copy(ew_v, sdeg.at[col_v], add=True)
        pltpu.sync_copy(one_v, scnt.at[col_v], add=True)
        return 0
    lax.fori_loop(0, WINDOWS, win, 0)

    plsc.subcore_barrier()
    pltpu.sync_copy(sdeg.at[nsl], degp_hbm.at[c, nsl])
    pltpu.sync_copy(scnt.at[nsl], cntp_hbm.at[c, nsl])


@functools.partial(
    pl.kernel,
    out_type=jax.ShapeDtypeStruct((NC, NP, EMBED), jnp.float32),
    mesh=_mesh,
    scratch_types=[
        pltpu.VMEM((WA,), jnp.int32),
        pltpu.VMEM((WA,), jnp.int32),
        pltpu.VMEM((WA,), jnp.int32),
        pltpu.VMEM((WA,), jnp.int32),
        pltpu.VMEM((WA,), jnp.float32),
        pltpu.VMEM((WA,), jnp.float32),
        pltpu.VMEM((WA, EMBED), jnp.float32),
        pltpu.VMEM((WA, EMBED), jnp.float32),
        pltpu.VMEM((WA, EMBED), jnp.float32),
        pltpu.VMEM((WA, EMBED), jnp.float32),
        pltpu.VMEM_SHARED((NP, EMBED), jnp.float32),
        pltpu.SemaphoreType.DMA,
        pltpu.SemaphoreType.DMA,
        pltpu.SemaphoreType.DMA,
        pltpu.SemaphoreType.DMA,
    ],
    compiler_params=_sc_params,
)
def _aggregate_sc(row_hbm, col_hbm, ew_hbm, g_hbm, sp_hbm,
                  row0, row1, col0, col1, ew0, ew1, rows0, rows1,
                  out0, out1, sacc, semg0, semg1, sems0, sems1):
    c = lax.axis_index("c")
    s = lax.axis_index("s")
    wid = _worker(c, s)
    iota16 = lax.iota(jnp.int32, LANES)
    zeros16 = jnp.zeros((LANES,), jnp.float32)
    zeros16i = jnp.zeros((LANES,), jnp.int32)
    rowb = [row0, row1]
    colb = [col0, col1]
    ewb_ = [ew0, ew1]
    rowsb = [rows0, rows1]
    outb = [out0, out1]
    semg = [semg0, semg1]
    sems = [sems0, sems1]
    dnums = lax.GatherDimensionNumbers(
        offset_dims=(), collapsed_slice_dims=(0,), start_index_map=(0,))

    def z_body(i, _):
        idx = jnp.full((LANES,), i, jnp.int32)
        plsc.store_scatter(out0, [idx, iota16], zeros16)
        plsc.store_scatter(out1, [idx, iota16], zeros16)
        return 0
    lax.fori_loop(0, WA, z_body, 0)

    def zc_body(i, _):
        col1[pl.ds(i * LANES, LANES)] = zeros16i
        return 0
    lax.fori_loop(0, WA // LANES, zc_body, 0)

    for t in range(NPT // ZROWS):
        dsl = pl.ds(pl.multiple_of(s * NPT + t * ZROWS, 8), ZROWS)
        pltpu.sync_copy(out0.at[pl.ds(0, ZROWS)], sacc.at[dsl])
    plsc.subcore_barrier()

    ebase = wid * EPW

    def lin_load(k, b):
        off = pl.multiple_of(ebase + k * WA, 8)
        esl = pl.ds(off, WA)
        pltpu.sync_copy(row_hbm.at[esl], rowb[b])
        pltpu.sync_copy(col_hbm.at[esl], colb[b])
        pltpu.sync_copy(ew_hbm.at[esl], ewb_[b])

    lin_load(0, 0)
    pltpu.async_copy(g_hbm.at[rowb[0]], rowsb[0], semg[0])
    pltpu.async_copy(outb[1], sacc.at[colb[1]], sems[1], add=True)

    def scale(b):
        def chunk(j, _):
            ewc = ewb_[b][pl.ds(j * LANES, LANES)]
            for t in range(LANES):
                e = j * LANES + t
                idx = jnp.full((LANES,), e, jnp.int32)
                sc = lax.gather(ewc, jnp.full((LANES, 1), t, jnp.int32),
                                dnums, (1,),
                                mode=lax.GatherScatterMode.PROMISE_IN_BOUNDS)
                r = plsc.load_gather(rowsb[b], [idx, iota16])
                plsc.store_scatter(outb[b], [idx, iota16], r * sc)
            return 0
        lax.fori_loop(0, WA // LANES, chunk, 0)

    def pair(p, _):
        for b in range(2):
            k = 2 * p + b
            nb = 1 - b
            kp = jnp.minimum(k + 1, WINDOWS_A - 1)
            pltpu.make_async_copy(outb[nb], sacc.at[colb[nb]],
                                  sems[nb]).wait()
            lin_load(kp, nb)
            pltpu.async_copy(g_hbm.at[rowb[nb]], rowsb[nb], semg[nb])
            pltpu.make_async_copy(g_hbm.at[rowb[b]], rowsb[b], semg[b]).wait()
            scale(b)
            pltpu.async_copy(outb[b], sacc.at[colb[b]], sems[b], add=True)
        return 0
    lax.fori_loop(0, WINDOWS_A // 2, pair, 0)

    pltpu.make_async_copy(outb[1], sacc.at[colb[1]], sems[1]).wait()
    pltpu.make_async_copy(g_hbm.at[rowb[0]], rowsb[0], semg[0]).wait()

    plsc.subcore_barrier()
    for t in range(NPT // ZROWS):
        dsl = pl.ds(pl.multiple_of(s * NPT + t * ZROWS, 8), ZROWS)
        pltpu.sync_copy(sacc.at[dsl], sp_hbm.at[c, dsl])


BLK = 2048
_softplus = jax.nn.softplus


def _full_spec(shape):
    return pl.BlockSpec(shape, lambda i: tuple(0 for _ in shape))


def _tc_prep_body(posq_ref, degp_ref, cntp_ref, wi_ref, bi_ref, wg1_ref,
                  g1_ref, self1_ref, dis_ref, cnt_ref, deg_ref):
    deg = degp_ref[0, :] + degp_ref[1, :] + 1.0
    cnt = cntp_ref[0, :] + cntp_ref[1, :] + 1.0
    dis = lax.rsqrt(deg)
    xe = _softplus(
        jnp.dot(posq_ref[...], wi_ref[...], preferred_element_type=jnp.float32)
        + bi_ref[...])
    hw1 = jnp.dot(xe, wg1_ref[...], preferred_element_type=jnp.float32)
    g1_ref[...] = dis[:, None] * hw1
    self1_ref[...] = hw1 / deg[:, None]
    dis_ref[...] = dis[:, None]
    cnt_ref[...] = cnt[:, None]
    deg_ref[...] = deg[:, None]


def _tc_prep(posq, degp, cntp, wi, bi, wg1):
    return pl.pallas_call(
        _tc_prep_body,
        grid=(NP // BLK,),
        in_specs=[
            pl.BlockSpec((BLK, 8), lambda i: (i, 0)),
            pl.BlockSpec((NC, BLK), lambda i: (0, i)),
            pl.BlockSpec((NC, BLK), lambda i: (0, i)),
            _full_spec((8, EMBED)),
            _full_spec((1, EMBED)),
            _full_spec((EMBED, EMBED)),
        ],
        out_specs=[
            pl.BlockSpec((BLK, EMBED), lambda i: (i, 0)),
            pl.BlockSpec((BLK, EMBED), lambda i: (i, 0)),
            pl.BlockSpec((BLK, 1), lambda i: (i, 0)),
            pl.BlockSpec((BLK, 1), lambda i: (i, 0)),
            pl.BlockSpec((BLK, 1), lambda i: (i, 0)),
        ],
        out_shape=[
            jax.ShapeDtypeStruct((NP, EMBED), jnp.float32),
            jax.ShapeDtypeStruct((NP, EMBED), jnp.float32),
            jax.ShapeDtypeStruct((NP, 1), jnp.float32),
            jax.ShapeDtypeStruct((NP, 1), jnp.float32),
            jax.ShapeDtypeStruct((NP, 1), jnp.float32),
        ],
    )(posq, degp, cntp, wi, bi, wg1)


def _tc_mid_body(sp_ref, dis_ref, cnt_ref, deg_ref, self1_ref, bg_ref,
                 wg2_ref, g2_ref, self2_ref):
    agg = sp_ref[0] + sp_ref[1]
    dis = dis_ref[...]
    x1 = _softplus((dis * agg + self1_ref[...]) / cnt_ref[...] + bg_ref[...])
    hw2 = jnp.dot(x1, wg2_ref[...], preferred_element_type=jnp.float32)
    g2_ref[...] = dis * hw2
    self2_ref[...] = hw2 / deg_ref[...]


def _tc_mid(sp, dis, cnt, deg, self1, bg1, wg2):
    return pl.pallas_call(
        _tc_mid_body,
        grid=(NP // BLK,),
        in_specs=[
            pl.BlockSpec((NC, BLK, EMBED), lambda i: (0, i, 0)),
            pl.BlockSpec((BLK, 1), lambda i: (i, 0)),
            pl.BlockSpec((BLK, 1), lambda i: (i, 0)),
            pl.BlockSpec((BLK, 1), lambda i: (i, 0)),
            pl.BlockSpec((BLK, EMBED), lambda i: (i, 0)),
            _full_spec((1, EMBED)),
            _full_spec((EMBED, EMBED)),
        ],
        out_specs=[
            pl.BlockSpec((BLK, EMBED), lambda i: (i, 0)),
            pl.BlockSpec((BLK, EMBED), lambda i: (i, 0)),
        ],
        out_shape=[
            jax.ShapeDtypeStruct((NP, EMBED), jnp.float32),
            jax.ShapeDtypeStruct((NP, EMBED), jnp.float32),
        ],
    )(sp, dis, cnt, deg, self1, bg1, wg2)


def _tc_final_body(sp_ref, dis_ref, cnt_ref, self2_ref, bg_ref, wp1_ref,
                   bp1_ref, wp2_ref, bp2_ref, sig_ref, out_ref):
    agg = sp_ref[0] + sp_ref[1]
    x2 = _softplus((dis_ref[...] * agg + self2_ref[...]) / cnt_ref[...]
                   + bg_ref[...])
    y = _softplus(
        jnp.dot(x2, wp1_ref[...], preferred_element_type=jnp.float32)
        + bp1_ref[...])
    sc = jnp.dot(y, wp2_ref[...], preferred_element_type=jnp.float32) \
        + bp2_ref[...]
    out_ref[...] = sc / sig_ref[...]


def _tc_final(sp, dis, cnt, self2, bg2, wp1, bp1, wp2q, bp2q, sig):
    return pl.pallas_call(
        _tc_final_body,
        grid=(NP // BLK,),
        in_specs=[
            pl.BlockSpec((NC, BLK, EMBED), lambda i: (0, i, 0)),
            pl.BlockSpec((BLK, 1), lambda i: (i, 0)),
            pl.BlockSpec((BLK, 1), lambda i: (i, 0)),
            pl.BlockSpec((BLK, EMBED), lambda i: (i, 0)),
            _full_spec((1, EMBED)),
            _full_spec((EMBED, EMBED)),
            _full_spec((1, EMBED)),
            _full_spec((EMBED, 8)),
            _full_spec((1, 8)),
            pl.BlockSpec((BLK, 1), lambda i: (i, 0)),
        ],
        out_specs=pl.BlockSpec((BLK, 8), lambda i: (i, 0)),
        out_shape=jax.ShapeDtypeStruct((NP, 8), jnp.float32),
    )(sp, dis, cnt, self2, bg2, wp1, bp1, wp2q, bp2q, sig)


def kernel(x, pos, edge_index, sigmas, W_init, b_init, W_g1, b_g1, W_g2,
           b_g2, W_p1, b_p1, W_p2, b_p2):
    del x
    row = edge_index[0].astype(jnp.int32)
    col = edge_index[1].astype(jnp.int32)

    padn = NP - N_NODES
    px = jnp.pad(pos[:, 0], (0, padn))
    py = jnp.pad(pos[:, 1], (0, padn))
    pz = jnp.pad(pos[:, 2], (0, padn))
    posq = jnp.pad(pos, ((0, padn), (0, 8 - pos.shape[1])))

    ew, degp, cntp = _edge_weights_sc(row, col, px, py, pz)

    wi = jnp.pad(W_init, ((0, 8 - W_init.shape[0]), (0, 0)))
    g1, self1, dis, cnt, deg = _tc_prep(posq, degp, cntp, wi,
                                        b_init[None, :], W_g1)

    sp1 = _aggregate_sc(row, col, ew, g1)
    g2, self2 = _tc_mid(sp1, dis, cnt, deg, self1, b_g1[None, :], W_g2)

    sp2 = _aggregate_sc(row, col, ew, g2)

    wp2q = jnp.pad(W_p2, ((0, 0), (0, 8 - W_p2.shape[1])))
    bp2q = jnp.pad(b_p2, (0, 8 - b_p2.shape[0]))[None, :]
    sig = jnp.pad(sigmas.reshape(-1), (0, padn), constant_values=1.0)
    out = _tc_final(sp2, dis, cnt, self2, b_g2[None, :], W_p1,
                    b_p1[None, :], wp2q, bp2q, sig[:, None])
    return out[:N_NODES, :3]


# speedup vs baseline: 1.1490x; 1.1490x over previous
import functools

import jax
import jax.numpy as jnp
from jax import lax
from jax.experimental import pallas as pl
from jax.experimental.pallas import tpu as pltpu
from jax.experimental.pallas import tpu_sc as plsc

N_NODES = 100000
N_EDGES = 3200000
EMBED = 16

NC = 2
NS = 16
NW = NC * NS
LANES = 16

NP = 102400
NPT = NP // NS
EPW = N_EDGES // NW
W = 4000
WINDOWS = EPW // W
WA = 400
WINDOWS_A = EPW // WA
ZROWS = 400

_mesh = plsc.VectorSubcoreMesh(
    core_axis_name="c", subcore_axis_name="s", num_cores=NC, num_subcores=NS)
_sc_params = pltpu.CompilerParams(needs_layout_passes=False,
                                  use_tc_tiling_on_sc=False)


def _worker(c, s):
    return s * NC + c


@functools.partial(
    pl.kernel,
    out_type=(
        jax.ShapeDtypeStruct((N_EDGES,), jnp.float32),
        jax.ShapeDtypeStruct((NC, NP), jnp.float32),
        jax.ShapeDtypeStruct((NC, NP), jnp.float32),
    ),
    mesh=_mesh,
    scratch_types=[
        pltpu.VMEM((W,), jnp.int32),
        pltpu.VMEM((W,), jnp.int32),
        pltpu.VMEM((W,), jnp.float32),
        pltpu.VMEM((W,), jnp.float32),
        pltpu.VMEM((W,), jnp.float32),
        pltpu.VMEM((W,), jnp.float32),
        pltpu.VMEM((W,), jnp.float32),
        pltpu.VMEM((W,), jnp.float32),
        pltpu.VMEM((W,), jnp.float32),
        pltpu.VMEM((W,), jnp.float32),
        pltpu.VMEM((NPT,), jnp.float32),
        pltpu.VMEM_SHARED((NP,), jnp.float32),
        pltpu.VMEM_SHARED((NP,), jnp.float32),
        pltpu.VMEM_SHARED((NP,), jnp.float32),
        pltpu.VMEM_SHARED((NP,), jnp.float32),
        pltpu.VMEM_SHARED((NP,), jnp.float32),
        pltpu.SemaphoreType.DMA,
    ],
    compiler_params=_sc_params,
)
def _edge_weights_sc(row_hbm, col_hbm, px_hbm, py_hbm, pz_hbm,
                     ew_hbm, degp_hbm, cntp_hbm,
                     row_v, col_v, xr, yr, zr, xc, yc, zc, ew_v, one_v, zb_v,
                     spx, spy, spz, sdeg, scnt, sem):
    c = lax.axis_index("c")
    s = lax.axis_index("s")
    wid = _worker(c, s)
    nsl = pl.ds(pl.multiple_of(s * NPT, 8), NPT)

    pltpu.sync_copy(px_hbm.at[nsl], spx.at[nsl])
    pltpu.sync_copy(py_hbm.at[nsl], spy.at[nsl])
    pltpu.sync_copy(pz_hbm.at[nsl], spz.at[nsl])

    zeros16 = jnp.zeros((LANES,), jnp.float32)
    ones16 = jnp.ones((LANES,), jnp.float32)

    def zb_body(i, _):
        zb_v[pl.ds(i * LANES, LANES)] = zeros16
        return 0
    lax.fori_loop(0, NPT // LANES, zb_body, 0)

    def one_body(i, _):
        one_v[pl.ds(i * LANES, LANES)] = ones16
        return 0
    lax.fori_loop(0, W // LANES, one_body, 0)

    pltpu.sync_copy(zb_v, sdeg.at[nsl])
    pltpu.sync_copy(zb_v, scnt.at[nsl])
    plsc.subcore_barrier()

    ebase = wid * EPW

    def win(k, _):
        b = pl.multiple_of(ebase + k * W, 8)
        esl = pl.ds(b, W)
        pltpu.sync_copy(row_hbm.at[esl], row_v)
        pltpu.sync_copy(col_hbm.at[esl], col_v)
        cps = [
            pltpu.async_copy(spx.at[row_v], xr, sem),
            pltpu.async_copy(spy.at[row_v], yr, sem),
            pltpu.async_copy(spz.at[row_v], zr, sem),
            pltpu.async_copy(spx.at[col_v], xc, sem),
            pltpu.async_copy(spy.at[col_v], yc, sem),
            pltpu.async_copy(spz.at[col_v], zc, sem),
        ]
        for cp in cps:
            cp.wait()

        def chunk(j, _):
            sl = pl.ds(j * LANES, LANES)
            dx = xr[sl] - xc[sl]
            dy = yr[sl] - yc[sl]
            dz = zr[sl] - zc[sl]
            sq = dx * dx + dy * dy + dz * dz
            bits = plsc.bitcast(sq, jnp.int32)
            y = plsc.bitcast(jnp.int32(0x5F3759DF) - (bits >> 1), jnp.float32)
            y = y * (1.5 - 0.5 * sq * y * y)
            y = y * (1.5 - 0.5 * sq * y * y)
            y = y * (1.5 - 0.5 * sq * y * y)
            ew = jnp.where(sq > 0.0, sq * y, 0.0)
            ew_v[sl] = ew
            return 0
        lax.fori_loop(0, W // LANES, chunk, 0)

        pltpu.sync_copy(ew_v, ew_hbm.at[esl])
        pltpu.sync_copy(ew_v, sdeg.at[col_v], add=True)
        pltpu.sync_copy(one_v, scnt.at[col_v], add=True)
        return 0
    lax.fori_loop(0, WINDOWS, win, 0)

    plsc.subcore_barrier()
    pltpu.sync_copy(sdeg.at[nsl], degp_hbm.at[c, nsl])
    pltpu.sync_copy(scnt.at[nsl], cntp_hbm.at[c, nsl])


@functools.partial(
    pl.kernel,
    out_type=jax.ShapeDtypeStruct((NC, NP, EMBED), jnp.float32),
    mesh=_mesh,
    scratch_types=[
        pltpu.VMEM((WA,), jnp.int32),
        pltpu.VMEM((WA,), jnp.int32),
        pltpu.VMEM((WA,), jnp.int32),
        pltpu.VMEM((WA,), jnp.int32),
        pltpu.VMEM((WA,), jnp.float32),
        pltpu.VMEM((WA,), jnp.float32),
        pltpu.VMEM((WA, EMBED), jnp.float32),
        pltpu.VMEM((WA, EMBED), jnp.float32),
        pltpu.VMEM((WA, EMBED), jnp.float32),
        pltpu.VMEM((WA, EMBED), jnp.float32),
        pltpu.VMEM_SHARED((NP, EMBED), jnp.float32),
        pltpu.SemaphoreType.DMA,
        pltpu.SemaphoreType.DMA,
        pltpu.SemaphoreType.DMA,
        pltpu.SemaphoreType.DMA,
    ],
    compiler_params=_sc_params,
)
def _aggregate_sc(row_hbm, col_hbm, ew_hbm, g_hbm, sp_hbm,
                  row0, row1, col0, col1, ew0, ew1, rows0, rows1,
                  out0, out1, sacc, semg0, semg1, sems0, sems1):
    c = lax.axis_index("c")
    s = lax.axis_index("s")
    wid = _worker(c, s)
    iota16 = lax.iota(jnp.int32, LANES)
    zeros16 = jnp.zeros((LANES,), jnp.float32)
    zeros16i = jnp.zeros((LANES,), jnp.int32)
    rowb = [row0, row1]
    colb = [col0, col1]
    ewb_ = [ew0, ew1]
    rowsb = [rows0, rows1]
    outb = [out0, out1]
    semg = [semg0, semg1]
    sems = [sems0, sems1]
    dnums = lax.GatherDimensionNumbers(
        offset_dims=(), collapsed_slice_dims=(0,), start_index_map=(0,))

    def z_body(i, _):
        idx = jnp.full((LANES,), i, jnp.int32)
        plsc.store_scatter(out0, [idx, iota16], zeros16)
        plsc.store_scatter(out1, [idx, iota16], zeros16)
        return 0
    lax.fori_loop(0, WA, z_body, 0)

    def zc_body(i, _):
        col1[pl.ds(i * LANES, LANES)] = zeros16i
        return 0
    lax.fori_loop(0, WA // LANES, zc_body, 0)

    for t in range(NPT // ZROWS):
        dsl = pl.ds(pl.multiple_of(s * NPT + t * ZROWS, 8), ZROWS)
        pltpu.sync_copy(out0.at[pl.ds(0, ZROWS)], sacc.at[dsl])
    plsc.subcore_barrier()

    ebase = wid * EPW

    def lin_load(k, b):
        off = pl.multiple_of(ebase + k * WA, 8)
        esl = pl.ds(off, WA)
        pltpu.sync_copy(row_hbm.at[esl], rowb[b])
        pltpu.sync_copy(col_hbm.at[esl], colb[b])
        pltpu.sync_copy(ew_hbm.at[esl], ewb_[b])

    lin_load(0, 0)
    pltpu.async_copy(g_hbm.at[rowb[0]], rowsb[0], semg[0])
    pltpu.async_copy(outb[1], sacc.at[colb[1]], sems[1], add=True)

    def scale(b):
        def chunk(j, _):
            ewc = ewb_[b][pl.ds(j * LANES, LANES)]
            for t in range(LANES):
                e = j * LANES + t
                idx = jnp.full((LANES,), e, jnp.int32)
                sc = lax.gather(ewc, jnp.full((LANES, 1), t, jnp.int32),
                                dnums, (1,),
                                mode=lax.GatherScatterMode.PROMISE_IN_BOUNDS)
                r = plsc.load_gather(rowsb[b], [idx, iota16])
                plsc.store_scatter(outb[b], [idx, iota16], r * sc)
            return 0
        lax.fori_loop(0, WA // LANES, chunk, 0)

    def pair(p, _):
        for b in range(2):
            k = 2 * p + b
            nb = 1 - b
            kp = jnp.minimum(k + 1, WINDOWS_A - 1)
            pltpu.make_async_copy(outb[nb], sacc.at[colb[nb]],
                                  sems[nb]).wait()
            lin_load(kp, nb)
            pltpu.async_copy(g_hbm.at[rowb[nb]], rowsb[nb], semg[nb])
            pltpu.make_async_copy(g_hbm.at[rowb[b]], rowsb[b], semg[b]).wait()
            scale(b)
            pltpu.async_copy(outb[b], sacc.at[colb[b]], sems[b], add=True)
        return 0
    lax.fori_loop(0, WINDOWS_A // 2, pair, 0)

    pltpu.make_async_copy(outb[1], sacc.at[colb[1]], sems[1]).wait()
    pltpu.make_async_copy(g_hbm.at[rowb[0]], rowsb[0], semg[0]).wait()

    plsc.subcore_barrier()
    for t in range(NPT // ZROWS):
        dsl = pl.ds(pl.multiple_of(s * NPT + t * ZROWS, 8), ZROWS)
        pltpu.sync_copy(sacc.at[dsl], sp_hbm.at[c, dsl])


NP8 = NP // 8
B8 = 256
_softplus = jax.nn.softplus


def _full_spec(shape):
    return pl.BlockSpec(shape, lambda i: tuple(0 for _ in shape))


def _expand16(x8):
    node = lax.broadcasted_iota(jnp.int32, (8, 128), 1) // EMBED
    rows = lax.broadcasted_iota(jnp.int32, (8, 128), 0)
    e = (node == rows).astype(jnp.float32)
    return jnp.dot(x8, e, preferred_element_type=jnp.float32)


def _tc_prep_body(posq_ref, degp_ref, cntp_ref, wi_ref, bi_ref, wg1_ref,
                  g1_ref, self1_ref, dis_ref, icnt_ref, ideg_ref):
    deg8 = degp_ref[0] + degp_ref[1] + 1.0
    cnt8 = cntp_ref[0] + cntp_ref[1] + 1.0
    dis16 = _expand16(lax.rsqrt(deg8))
    ideg16 = _expand16(1.0 / deg8)
    icnt16 = _expand16(1.0 / cnt8)
    xe = _softplus(
        jnp.dot(posq_ref[...], wi_ref[...], preferred_element_type=jnp.float32)
        + bi_ref[...])
    hw1 = jnp.dot(xe, wg1_ref[...], preferred_element_type=jnp.float32)
    g1_ref[...] = dis16 * hw1
    self1_ref[...] = hw1 * ideg16
    dis_ref[...] = dis16
    icnt_ref[...] = icnt16
    ideg_ref[...] = ideg16


def _tc_prep(posq, degp, cntp, wi, bi, wg1):
    node_sds = jax.ShapeDtypeStruct((NP8, 128), jnp.float32)
    return pl.pallas_call(
        _tc_prep_body,
        grid=(NP8 // B8,),
        in_specs=[
            pl.BlockSpec((B8, 64), lambda i: (i, 0)),
            pl.BlockSpec((NC, B8, 8), lambda i: (0, i, 0)),
            pl.BlockSpec((NC, B8, 8), lambda i: (0, i, 0)),
            _full_spec((64, 128)),
            _full_spec((1, 128)),
            _full_spec((128, 128)),
        ],
        out_specs=[pl.BlockSpec((B8, 128), lambda i: (i, 0))] * 5,
        out_shape=[node_sds] * 5,
    )(posq, degp, cntp, wi, bi, wg1)


def _tc_mid_body(sp_ref, dis_ref, icnt_ref, ideg_ref, self1_ref, bg_ref,
                 wg2_ref, g2_ref, self2_ref):
    agg = sp_ref[0] + sp_ref[1]
    dis16 = dis_ref[...]
    x1 = _softplus((dis16 * agg + self1_ref[...]) * icnt_ref[...]
                   + bg_ref[...])
    hw2 = jnp.dot(x1, wg2_ref[...], preferred_element_type=jnp.float32)
    g2_ref[...] = dis16 * hw2
    self2_ref[...] = hw2 * ideg_ref[...]


def _tc_mid(sp, dis, icnt, ideg, self1, bg1, wg2):
    node_sds = jax.ShapeDtypeStruct((NP8, 128), jnp.float32)
    return pl.pallas_call(
        _tc_mid_body,
        grid=(NP8 // B8,),
        in_specs=[
            pl.BlockSpec((NC, B8, 128), lambda i: (0, i, 0)),
            pl.BlockSpec((B8, 128), lambda i: (i, 0)),
            pl.BlockSpec((B8, 128), lambda i: (i, 0)),
            pl.BlockSpec((B8, 128), lambda i: (i, 0)),
            pl.BlockSpec((B8, 128), lambda i: (i, 0)),
            _full_spec((1, 128)),
            _full_spec((128, 128)),
        ],
        out_specs=[pl.BlockSpec((B8, 128), lambda i: (i, 0))] * 2,
        out_shape=[node_sds] * 2,
    )(sp, dis, icnt, ideg, self1, bg1, wg2)


def _tc_final_body(sp_ref, dis_ref, icnt_ref, self2_ref, bg_ref, wp1_ref,
                   bp1_ref, wp2_ref, bp2_ref, sig_ref, out_ref):
    agg = sp_ref[0] + sp_ref[1]
    x2 = _softplus((dis_ref[...] * agg + self2_ref[...]) * icnt_ref[...]
                   + bg_ref[...])
    y = _softplus(
        jnp.dot(x2, wp1_ref[...], preferred_element_type=jnp.float32)
        + bp1_ref[...])
    sc = jnp.dot(y, wp2_ref[...], preferred_element_type=jnp.float32) \
        + bp2_ref[...]
    out_ref[...] = sc / sig_ref[...]


def _tc_final(sp, dis, icnt, self2, bg2, wp1, bp1, wp2q, bp2q, sig):
    return pl.pallas_call(
        _tc_final_body,
        grid=(NP8 // B8,),
        in_specs=[
            pl.BlockSpec((NC, B8, 128), lambda i: (0, i, 0)),
            pl.BlockSpec((B8, 128), lambda i: (i, 0)),
            pl.BlockSpec((B8, 128), lambda i: (i, 0)),
            pl.BlockSpec((B8, 128), lambda i: (i, 0)),
            _full_spec((1, 128)),
            _full_spec((128, 128)),
            _full_spec((1, 128)),
            _full_spec((128, 64)),
            _full_spec((1, 64)),
            pl.BlockSpec((B8, 64), lambda i: (i, 0)),
        ],
        out_specs=pl.BlockSpec((B8, 64), lambda i: (i, 0)),
        out_shape=jax.ShapeDtypeStruct((NP8, 64), jnp.float32),
    )(sp, dis, icnt, self2, bg2, wp1, bp1, wp2q, bp2q, sig)


def kernel(x, pos, edge_index, sigmas, W_init, b_init, W_g1, b_g1, W_g2,
           b_g2, W_p1, b_p1, W_p2, b_p2):
    del x
    row = edge_index[0].astype(jnp.int32)
    col = edge_index[1].astype(jnp.int32)

    padn = NP - N_NODES
    px = jnp.pad(pos[:, 0], (0, padn))
    py = jnp.pad(pos[:, 1], (0, padn))
    pz = jnp.pad(pos[:, 2], (0, padn))
    posq = jnp.pad(pos, ((0, padn), (0, 8 - pos.shape[1])))

    ew, degp, cntp = _edge_weights_sc(row, col, px, py, pz)

    eye8 = jnp.eye(8, dtype=jnp.float32)
    wi = jnp.kron(eye8, jnp.pad(W_init, ((0, 8 - W_init.shape[0]), (0, 0))))
    wg1 = jnp.kron(eye8, W_g1)
    wg2 = jnp.kron(eye8, W_g2)
    wp1 = jnp.kron(eye8, W_p1)
    wp2 = jnp.kron(eye8, jnp.pad(W_p2, ((0, 0), (0, 8 - W_p2.shape[1]))))
    bi = jnp.tile(b_init, 8)[None, :]
    bg1 = jnp.tile(b_g1, 8)[None, :]
    bg2 = jnp.tile(b_g2, 8)[None, :]
    bp1 = jnp.tile(b_p1, 8)[None, :]
    bp2 = jnp.tile(jnp.pad(b_p2, (0, 8 - b_p2.shape[0])), 8)[None, :]

    g1, self1, dis, icnt, ideg = _tc_prep(
        posq.reshape(NP8, 64), degp.reshape(NC, NP8, 8),
        cntp.reshape(NC, NP8, 8), wi, bi, wg1)

    sp1 = _aggregate_sc(row, col, ew, g1.reshape(NP, EMBED))
    g2, self2 = _tc_mid(sp1.reshape(NC, NP8, 128), dis, icnt, ideg, self1,
                        bg1, wg2)

    sp2 = _aggregate_sc(row, col, ew, g2.reshape(NP, EMBED))

    sig = jnp.pad(sigmas.reshape(-1), (0, padn), constant_values=1.0)
    sig64 = jnp.broadcast_to(sig[:, None], (NP, 8)).reshape(NP8, 64)
    out = _tc_final(sp2.reshape(NC, NP8, 128), dis, icnt, self2, bg2, wp1,
                    bp1, wp2, bp2, sig64)
    return out.reshape(NP, 8)[:N_NODES, :3]
